# baseline (speedup 1.0000x reference)
import functools

import jax
import jax.numpy as jnp
from jax import lax
from jax.experimental import pallas as pl
from jax.experimental.pallas import tpu as pltpu
from jax.experimental.pallas import tpu_sc as plsc

NC = 2
NS = 16
NW = NC * NS
LANES = 16
CHUNK = 128
ACC_ROWS = 10240
BLK = 400
C0_NUM = 5
C0_DEN = 8


def _sc_mesh():
    return plsc.VectorSubcoreMesh(core_axis_name="c", subcore_axis_name="s")


def _degree_hist(col2d, ncw):
    @functools.partial(
        pl.kernel,
        out_type=jax.ShapeDtypeStruct((NC, ACC_ROWS), jnp.float32),
        mesh=_sc_mesh(),
        scratch_types=[
            pltpu.VMEM((ncw, CHUNK), jnp.int32),
            pltpu.VMEM((CHUNK,), jnp.float32),
            pltpu.VMEM((ACC_ROWS // NS,), jnp.float32),
            pltpu.VMEM_SHARED((ACC_ROWS,), jnp.float32),
            pltpu.SemaphoreType.DMA,
        ],
    )
    def k(col_hbm, out_hbm, colbuf, ones, zbuf, hist, ssem):
        cid = lax.axis_index("c")
        sid = lax.axis_index("s")
        stripe = ACC_ROWS // NS
        hbase = pl.multiple_of(sid * stripe, 8)
        wid = cid * NS + sid
        cbase = pl.multiple_of(wid * ncw, 8)

        def zfill(i, c):
            zbuf[pl.ds(i * LANES, LANES)] = jnp.zeros((LANES,), jnp.float32)
            return c

        lax.fori_loop(0, stripe // LANES, zfill, 0)

        def ofill(i, c):
            ones[pl.ds(i * LANES, LANES)] = jnp.ones((LANES,), jnp.float32)
            return c

        lax.fori_loop(0, CHUNK // LANES, ofill, 0)
        pltpu.sync_copy(col_hbm.at[pl.ds(cbase, ncw)], colbuf)
        pltpu.sync_copy(zbuf, hist.at[pl.ds(hbase, stripe)])
        plsc.subcore_barrier()

        def body(j, c):
            pltpu.async_copy(ones, hist.at[colbuf.at[j]], ssem, add=True)
            return c

        lax.fori_loop(0, ncw, body, 0)

        def drain(j, c):
            pltpu.make_async_copy(ones, hist.at[colbuf.at[j]], ssem).wait()
            return c

        lax.fori_loop(0, ncw, drain, 0)
        plsc.subcore_barrier()
        pltpu.sync_copy(hist.at[pl.ds(hbase, stripe)],
                        out_hbm.at[cid, pl.ds(hbase, stripe)])

    return k(col2d)


def _aggregate(u, rc3d, ncw):
    D = u.shape[1]
    ncw0 = 2 * ((2 * ncw * C0_NUM) // (C0_DEN * 2))
    ncw1 = 2 * ncw - ncw0

    @functools.partial(
        pl.kernel,
        out_type=jax.ShapeDtypeStruct((NC, ACC_ROWS, D), jnp.float32),
        mesh=_sc_mesh(),
        scratch_types=[
            pltpu.VMEM((2, 2, CHUNK), jnp.int32),
            pltpu.VMEM((2, CHUNK, D), jnp.float32),
            pltpu.VMEM_SHARED((ACC_ROWS, D), jnp.float32),
            [pltpu.SemaphoreType.DMA] * 2,
            [pltpu.SemaphoreType.DMA] * 2,
        ],
    )
    def k(u_hbm, rc_hbm, out_hbm, rcbuf, gbuf, acc, isems, gsems):
        cid = lax.axis_index("c")
        sid = lax.axis_index("s")
        stripe = ACC_ROWS // NS
        sbase = pl.multiple_of(sid * stripe, 8)
        nch = lax.select(cid == 0, ncw0, ncw1)
        base = lax.select(cid == 0, sid * ncw0, NS * ncw0 + sid * ncw1)

        def zfill(i, c):
            r = i // (D // LANES)
            q = lax.rem(i, D // LANES)
            gbuf[0, r, pl.ds(q * LANES, LANES)] = jnp.zeros(
                (LANES,), jnp.float32)
            return c

        lax.fori_loop(0, CHUNK * D // LANES, zfill, 0)
        for i in range(stripe // CHUNK):
            pltpu.sync_copy(gbuf.at[0],
                            acc.at[pl.ds(sbase + i * CHUNK, CHUNK)])
        plsc.subcore_barrier()

        pltpu.async_copy(rc_hbm.at[base], rcbuf.at[0], isems[0])
        pltpu.async_copy(rc_hbm.at[base + 1], rcbuf.at[1], isems[1])
        pltpu.make_async_copy(rc_hbm.at[base], rcbuf.at[0], isems[0]).wait()
        pltpu.async_copy(u_hbm.at[rcbuf.at[0, 0]], gbuf.at[0], gsems[0])

        def body(t, c):
            for b in range(2):
                j = t * 2 + b
                b2 = 1 - b

                @pl.when(j + 1 < nch)
                def _():
                    pltpu.make_async_copy(
                        rc_hbm.at[base + j + 1], rcbuf.at[b2],
                        isems[b2]).wait()
                    pltpu.async_copy(
                        u_hbm.at[rcbuf.at[b2, 0]], gbuf.at[b2], gsems[b2])

                pltpu.make_async_copy(
                    u_hbm.at[rcbuf.at[b, 0]], gbuf.at[b], gsems[b]).wait()
                pltpu.sync_copy(gbuf.at[b], acc.at[rcbuf.at[b, 1]], add=True)

                @pl.when(j + 2 < nch)
                def _():
                    pltpu.async_copy(
                        rc_hbm.at[base + j + 2], rcbuf.at[b], isems[b])
            return c

        lax.fori_loop(0, nch // 2, body, 0)
        plsc.subcore_barrier()
        pltpu.sync_copy(acc.at[pl.ds(sbase, stripe)],
                        out_hbm.at[cid, pl.ds(sbase, stripe)])

    return k(u, rc3d)


def _tc_dis(hist):
    nr = hist.shape[1]

    def body(h_ref, o_ref):
        h = h_ref[...]
        o_ref[...] = lax.rsqrt(h[0] + h[1] + 1.0)[:, None]

    return pl.pallas_call(
        body,
        out_shape=jax.ShapeDtypeStruct((nr, 1), jnp.float32),
    )(hist)


def _tc_first(x, W, dis):
    n, din = x.shape
    dh = W.shape[1]

    def body(x_ref, w_ref, d_ref, o_ref):
        o_ref[...] = jnp.dot(
            x_ref[...], w_ref[...], preferred_element_type=jnp.float32
        ) * d_ref[...]

    return pl.pallas_call(
        body,
        grid=(n // BLK,),
        in_specs=[
            pl.BlockSpec((BLK, din), lambda i: (i, 0)),
            pl.BlockSpec((din, dh), lambda i: (0, 0)),
            pl.BlockSpec((BLK, 1), lambda i: (i, 0)),
        ],
        out_specs=pl.BlockSpec((BLK, dh), lambda i: (i, 0)),
        out_shape=jax.ShapeDtypeStruct((n, dh), jnp.float32),
    )(x, W, dis)


def _tc_mid(S, u_prev, dis, b, W):
    n, dh = u_prev.shape
    do = W.shape[1]

    def body(s_ref, u_ref, d_ref, b_ref, w_ref, o_ref):
        d = d_ref[...]
        a = jnp.maximum(
            (s_ref[0] + s_ref[1] + u_ref[...]) * d + b_ref[...], 0.0)
        o_ref[...] = jnp.dot(
            a, w_ref[...], preferred_element_type=jnp.float32) * d

    return pl.pallas_call(
        body,
        grid=(n // BLK,),
        in_specs=[
            pl.BlockSpec((NC, BLK, dh), lambda i: (0, i, 0)),
            pl.BlockSpec((BLK, dh), lambda i: (i, 0)),
            pl.BlockSpec((BLK, 1), lambda i: (i, 0)),
            pl.BlockSpec((1, dh), lambda i: (0, 0)),
            pl.BlockSpec((dh, do), lambda i: (0, 0)),
        ],
        out_specs=pl.BlockSpec((BLK, do), lambda i: (i, 0)),
        out_shape=jax.ShapeDtypeStruct((n, do), jnp.float32),
    )(S, u_prev, dis, b, W)


def _tc_last(S, u_prev, dis, b, Wc, bc):
    n, dh = u_prev.shape
    do = Wc.shape[1]

    def body(s_ref, u_ref, d_ref, b_ref, w_ref, bc_ref, o_ref):
        a = jnp.maximum(
            (s_ref[0] + s_ref[1] + u_ref[...]) * d_ref[...] + b_ref[...], 0.0)
        o_ref[...] = jnp.dot(
            a, w_ref[...], preferred_element_type=jnp.float32) + bc_ref[...]

    return pl.pallas_call(
        body,
        grid=(n // BLK,),
        in_specs=[
            pl.BlockSpec((NC, BLK, dh), lambda i: (0, i, 0)),
            pl.BlockSpec((BLK, dh), lambda i: (i, 0)),
            pl.BlockSpec((BLK, 1), lambda i: (i, 0)),
            pl.BlockSpec((1, dh), lambda i: (0, 0)),
            pl.BlockSpec((dh, do), lambda i: (0, 0)),
            pl.BlockSpec((1, do), lambda i: (0, 0)),
        ],
        out_specs=pl.BlockSpec((BLK, do), lambda i: (i, 0)),
        out_shape=jax.ShapeDtypeStruct((n, do), jnp.float32),
    )(S, u_prev, dis, b, Wc, bc)


def kernel(x, edge_index, W1, b1, W2, b2, Wc, bc):
    n, _ = x.shape
    e = edge_index.shape[1]
    row = edge_index[0].astype(jnp.int32)
    col = edge_index[1].astype(jnp.int32)
    block = NW * CHUNK * 8
    epad = ((e + block - 1) // block) * block
    npad = epad - e
    if npad:
        row = jnp.concatenate([row, jnp.zeros((npad,), jnp.int32)])
        col = jnp.concatenate([col, jnp.full((npad,), n, jnp.int32)])
    rc3d = jnp.stack(
        [row.reshape(-1, CHUNK), col.reshape(-1, CHUNK)], axis=1)
    col2d = col.reshape(-1, CHUNK)
    ncw = col2d.shape[0] // NW

    hist = _degree_hist(col2d, ncw)
    dis = _tc_dis(hist)[:n]
    u1 = _tc_first(x, W1, dis)
    s1 = _aggregate(u1, rc3d, ncw)
    u2 = _tc_mid(s1[:, :n], u1, dis, b1.reshape(1, -1), W2)
    s2 = _aggregate(u2, rc3d, ncw)
    return _tc_last(s2[:, :n], u2, dis, b2.reshape(1, -1), Wc,
                    bc.reshape(1, -1))


# speedup vs baseline: 1.0131x; 1.0131x over previous
import functools

import jax
import jax.numpy as jnp
from jax import lax
from jax.experimental import pallas as pl
from jax.experimental.pallas import tpu as pltpu
from jax.experimental.pallas import tpu_sc as plsc

NC = 2
NS = 16
NW = NC * NS
LANES = 16
CHUNK = 128
ACC_ROWS = 10240
BLK = 400
C0_NUM = 7
C0_DEN = 8


def _sc_mesh():
    return plsc.VectorSubcoreMesh(core_axis_name="c", subcore_axis_name="s")


def _degree_hist(col2d, ncw):
    @functools.partial(
        pl.kernel,
        out_type=jax.ShapeDtypeStruct((NC, ACC_ROWS), jnp.float32),
        mesh=_sc_mesh(),
        scratch_types=[
            pltpu.VMEM((ncw, CHUNK), jnp.int32),
            pltpu.VMEM((CHUNK,), jnp.float32),
            pltpu.VMEM((ACC_ROWS // NS,), jnp.float32),
            pltpu.VMEM_SHARED((ACC_ROWS,), jnp.float32),
            pltpu.SemaphoreType.DMA,
        ],
    )
    def k(col_hbm, out_hbm, colbuf, ones, zbuf, hist, ssem):
        cid = lax.axis_index("c")
        sid = lax.axis_index("s")
        stripe = ACC_ROWS // NS
        hbase = pl.multiple_of(sid * stripe, 8)
        wid = cid * NS + sid
        cbase = pl.multiple_of(wid * ncw, 8)

        def zfill(i, c):
            zbuf[pl.ds(i * LANES, LANES)] = jnp.zeros((LANES,), jnp.float32)
            return c

        lax.fori_loop(0, stripe // LANES, zfill, 0)

        def ofill(i, c):
            ones[pl.ds(i * LANES, LANES)] = jnp.ones((LANES,), jnp.float32)
            return c

        lax.fori_loop(0, CHUNK // LANES, ofill, 0)
        pltpu.sync_copy(col_hbm.at[pl.ds(cbase, ncw)], colbuf)
        pltpu.sync_copy(zbuf, hist.at[pl.ds(hbase, stripe)])
        plsc.subcore_barrier()

        def body(j, c):
            pltpu.async_copy(ones, hist.at[colbuf.at[j]], ssem, add=True)
            return c

        lax.fori_loop(0, ncw, body, 0)

        def drain(j, c):
            pltpu.make_async_copy(ones, hist.at[colbuf.at[j]], ssem).wait()
            return c

        lax.fori_loop(0, ncw, drain, 0)
        plsc.subcore_barrier()
        pltpu.sync_copy(hist.at[pl.ds(hbase, stripe)],
                        out_hbm.at[cid, pl.ds(hbase, stripe)])

    return k(col2d)


def _aggregate(u, rc3d, ncw):
    D = u.shape[1]
    ncw0 = 2 * ((2 * ncw * C0_NUM) // (C0_DEN * 2))
    ncw1 = 2 * ncw - ncw0

    @functools.partial(
        pl.kernel,
        out_type=jax.ShapeDtypeStruct((NC, ACC_ROWS, D), jnp.float32),
        mesh=_sc_mesh(),
        scratch_types=[
            pltpu.VMEM((2, 2, CHUNK), jnp.int32),
            pltpu.VMEM((2, CHUNK, D), jnp.float32),
            pltpu.VMEM_SHARED((ACC_ROWS, D), jnp.float32),
            [pltpu.SemaphoreType.DMA] * 2,
            [pltpu.SemaphoreType.DMA] * 2,
        ],
    )
    def k(u_hbm, rc_hbm, out_hbm, rcbuf, gbuf, acc, isems, gsems):
        cid = lax.axis_index("c")
        sid = lax.axis_index("s")
        stripe = ACC_ROWS // NS
        sbase = pl.multiple_of(sid * stripe, 8)
        nch = lax.select(cid == 0, ncw0, ncw1)
        base = lax.select(cid == 0, sid * ncw0, NS * ncw0 + sid * ncw1)

        def zfill(i, c):
            r = i // (D // LANES)
            q = lax.rem(i, D // LANES)
            gbuf[0, r, pl.ds(q * LANES, LANES)] = jnp.zeros(
                (LANES,), jnp.float32)
            return c

        lax.fori_loop(0, CHUNK * D // LANES, zfill, 0)
        for i in range(stripe // CHUNK):
            pltpu.sync_copy(gbuf.at[0],
                            acc.at[pl.ds(sbase + i * CHUNK, CHUNK)])
        plsc.subcore_barrier()

        pltpu.async_copy(rc_hbm.at[base], rcbuf.at[0], isems[0])
        pltpu.async_copy(rc_hbm.at[base + 1], rcbuf.at[1], isems[1])
        pltpu.make_async_copy(rc_hbm.at[base], rcbuf.at[0], isems[0]).wait()
        pltpu.async_copy(u_hbm.at[rcbuf.at[0, 0]], gbuf.at[0], gsems[0])

        def body(t, c):
            for b in range(2):
                j = t * 2 + b
                b2 = 1 - b

                @pl.when(j + 1 < nch)
                def _():
                    pltpu.make_async_copy(
                        rc_hbm.at[base + j + 1], rcbuf.at[b2],
                        isems[b2]).wait()
                    pltpu.async_copy(
                        u_hbm.at[rcbuf.at[b2, 0]], gbuf.at[b2], gsems[b2])

                pltpu.make_async_copy(
                    u_hbm.at[rcbuf.at[b, 0]], gbuf.at[b], gsems[b]).wait()
                pltpu.sync_copy(gbuf.at[b], acc.at[rcbuf.at[b, 1]], add=True)

                @pl.when(j + 2 < nch)
                def _():
                    pltpu.async_copy(
                        rc_hbm.at[base + j + 2], rcbuf.at[b], isems[b])
            return c

        lax.fori_loop(0, nch // 2, body, 0)
        plsc.subcore_barrier()
        pltpu.sync_copy(acc.at[pl.ds(sbase, stripe)],
                        out_hbm.at[cid, pl.ds(sbase, stripe)])

    return k(u, rc3d)


def _tc_dis(hist):
    nr = hist.shape[1]

    def body(h_ref, o_ref):
        h = h_ref[...]
        o_ref[...] = lax.rsqrt(h[0] + h[1] + 1.0)[:, None]

    return pl.pallas_call(
        body,
        out_shape=jax.ShapeDtypeStruct((nr, 1), jnp.float32),
    )(hist)


def _tc_first(x, W, dis):
    n, din = x.shape
    dh = W.shape[1]

    def body(x_ref, w_ref, d_ref, o_ref):
        o_ref[...] = jnp.dot(
            x_ref[...], w_ref[...], preferred_element_type=jnp.float32
        ) * d_ref[...]

    return pl.pallas_call(
        body,
        grid=(n // BLK,),
        in_specs=[
            pl.BlockSpec((BLK, din), lambda i: (i, 0)),
            pl.BlockSpec((din, dh), lambda i: (0, 0)),
            pl.BlockSpec((BLK, 1), lambda i: (i, 0)),
        ],
        out_specs=pl.BlockSpec((BLK, dh), lambda i: (i, 0)),
        out_shape=jax.ShapeDtypeStruct((n, dh), jnp.float32),
    )(x, W, dis)


def _tc_mid(S, u_prev, dis, b, W):
    n, dh = u_prev.shape
    do = W.shape[1]

    def body(s_ref, u_ref, d_ref, b_ref, w_ref, o_ref):
        d = d_ref[...]
        a = jnp.maximum(
            (s_ref[0] + s_ref[1] + u_ref[...]) * d + b_ref[...], 0.0)
        o_ref[...] = jnp.dot(
            a, w_ref[...], preferred_element_type=jnp.float32) * d

    return pl.pallas_call(
        body,
        grid=(n // BLK,),
        in_specs=[
            pl.BlockSpec((NC, BLK, dh), lambda i: (0, i, 0)),
            pl.BlockSpec((BLK, dh), lambda i: (i, 0)),
            pl.BlockSpec((BLK, 1), lambda i: (i, 0)),
            pl.BlockSpec((1, dh), lambda i: (0, 0)),
            pl.BlockSpec((dh, do), lambda i: (0, 0)),
        ],
        out_specs=pl.BlockSpec((BLK, do), lambda i: (i, 0)),
        out_shape=jax.ShapeDtypeStruct((n, do), jnp.float32),
    )(S, u_prev, dis, b, W)


def _tc_last(S, u_prev, dis, b, Wc, bc):
    n, dh = u_prev.shape
    do = Wc.shape[1]

    def body(s_ref, u_ref, d_ref, b_ref, w_ref, bc_ref, o_ref):
        a = jnp.maximum(
            (s_ref[0] + s_ref[1] + u_ref[...]) * d_ref[...] + b_ref[...], 0.0)
        o_ref[...] = jnp.dot(
            a, w_ref[...], preferred_element_type=jnp.float32) + bc_ref[...]

    return pl.pallas_call(
        body,
        grid=(n // BLK,),
        in_specs=[
            pl.BlockSpec((NC, BLK, dh), lambda i: (0, i, 0)),
            pl.BlockSpec((BLK, dh), lambda i: (i, 0)),
            pl.BlockSpec((BLK, 1), lambda i: (i, 0)),
            pl.BlockSpec((1, dh), lambda i: (0, 0)),
            pl.BlockSpec((dh, do), lambda i: (0, 0)),
            pl.BlockSpec((1, do), lambda i: (0, 0)),
        ],
        out_specs=pl.BlockSpec((BLK, do), lambda i: (i, 0)),
        out_shape=jax.ShapeDtypeStruct((n, do), jnp.float32),
    )(S, u_prev, dis, b, Wc, bc)


def kernel(x, edge_index, W1, b1, W2, b2, Wc, bc):
    n, _ = x.shape
    e = edge_index.shape[1]
    row = edge_index[0].astype(jnp.int32)
    col = edge_index[1].astype(jnp.int32)
    block = NW * CHUNK * 8
    epad = ((e + block - 1) // block) * block
    npad = epad - e
    if npad:
        row = jnp.concatenate([row, jnp.zeros((npad,), jnp.int32)])
        col = jnp.concatenate([col, jnp.full((npad,), n, jnp.int32)])
    rc3d = jnp.stack(
        [row.reshape(-1, CHUNK), col.reshape(-1, CHUNK)], axis=1)
    col2d = col.reshape(-1, CHUNK)
    ncw = col2d.shape[0] // NW

    hist = _degree_hist(col2d, ncw)
    dis = _tc_dis(hist)[:n]
    u1 = _tc_first(x, W1, dis)
    s1 = _aggregate(u1, rc3d, ncw)
    u2 = _tc_mid(s1[:, :n], u1, dis, b1.reshape(1, -1), W2)
    s2 = _aggregate(u2, rc3d, ncw)
    return _tc_last(s2[:, :n], u2, dis, b2.reshape(1, -1), Wc,
                    bc.reshape(1, -1))
